# baseline (speedup 1.0000x reference)
import jax
import jax.numpy as jnp
import numpy as np
from jax import lax
from jax.experimental import pallas as pl
from jax.experimental.pallas import tpu as pltpu

N_BINS = 15
N_ROWS = 50000
N_COLS = 1000
COL_BLOCK = 6144
GRID = (N_ROWS + COL_BLOCK - 1) // COL_BLOCK

_BOUNDS = np.array(
    [0x00000000, 0x3D888889, 0x3E088889, 0x3E4CCCCE, 0x3E888889, 0x3EAAAAAB,
     0x3ECCCCCE, 0x3EEEEEF0, 0x3F088889, 0x3F19999A, 0x3F2AAAAB, 0x3F3BBBBC,
     0x3F4CCCCE, 0x3F5DDDDF, 0x3F6EEEF0, 0x3F800000],
    dtype=np.uint32,
).view(np.float32)


def _ece_body(x_ref, lbl_ref, out_ref, cnt_ref, csum_ref, asum_ref):
    step = pl.program_id(0)

    @pl.when(step == 0)
    def _init():
        cnt_ref[...] = jnp.zeros_like(cnt_ref)
        csum_ref[...] = jnp.zeros_like(csum_ref)
        asum_ref[...] = jnp.zeros_like(asum_ref)

    x = x_ref[...]
    m = jnp.max(x, axis=0, keepdims=True)
    s = jnp.sum(jnp.exp(x - m), axis=0, keepdims=True)
    conf = 1.0 / s

    ii = lax.broadcasted_iota(jnp.int32, x.shape, 0)
    pred = jnp.min(jnp.where(x == m, ii, N_COLS), axis=0, keepdims=True)
    acc = (pred == lbl_ref[...]).astype(jnp.float32)

    subl = lax.broadcasted_iota(jnp.int32, (2 * 8, 1), 0)
    lo = jnp.full((2 * 8, 1), jnp.inf, jnp.float32)
    up = jnp.full((2 * 8, 1), jnp.inf, jnp.float32)
    for i in range(N_BINS):
        lo = jnp.where(subl == i, _BOUNDS[i], lo)
        up = jnp.where(subl == i, _BOUNDS[i + 1], up)

    sidx = step * COL_BLOCK + lax.broadcasted_iota(jnp.int32, (1, COL_BLOCK), 1)
    valid = sidx < N_ROWS

    confb = jnp.broadcast_to(conf, (2 * 8, COL_BLOCK))
    accb = jnp.broadcast_to(acc, (2 * 8, COL_BLOCK))
    inb = (confb > lo) & (confb <= up) & valid
    zero = jnp.zeros((2 * 8, COL_BLOCK), jnp.float32)
    cnt_ref[...] += jnp.where(inb, 1.0, zero)
    csum_ref[...] += jnp.where(inb, confb, zero)
    asum_ref[...] += jnp.where(inb, accb, zero)

    @pl.when(step == GRID - 1)
    def _combine():
        tot_cnt = jnp.sum(cnt_ref[...], axis=1, keepdims=True)
        tot_conf = jnp.sum(csum_ref[...], axis=1, keepdims=True)
        tot_acc = jnp.sum(asum_ref[...], axis=1, keepdims=True)
        safe = jnp.maximum(tot_cnt, 1.0)
        contrib = jnp.abs(tot_conf / safe - tot_acc / safe) * (
            tot_cnt * (1.0 / N_ROWS)
        )
        contrib = jnp.where(tot_cnt > 0.0, contrib, 0.0)
        out_ref[...] = jnp.broadcast_to(jnp.sum(contrib), (8, 128))


@jax.jit
def kernel(logits, labels):
    xt = logits.T
    lbl = labels.astype(jnp.int32).reshape(1, N_ROWS)
    out = pl.pallas_call(
        _ece_body,
        grid=(GRID,),
        in_specs=[
            pl.BlockSpec((N_COLS, COL_BLOCK), lambda i: (0, i)),
            pl.BlockSpec((1, COL_BLOCK), lambda i: (0, i)),
        ],
        out_specs=pl.BlockSpec((8, 128), lambda i: (0, 0)),
        out_shape=jax.ShapeDtypeStruct((8, 128), jnp.float32),
        scratch_shapes=[
            pltpu.VMEM((2 * 8, COL_BLOCK), jnp.float32),
            pltpu.VMEM((2 * 8, COL_BLOCK), jnp.float32),
            pltpu.VMEM((2 * 8, COL_BLOCK), jnp.float32),
        ],
        compiler_params=pltpu.CompilerParams(
            dimension_semantics=("arbitrary",),
        ),
    )(xt, lbl)
    return out[0, 0].reshape(1)


# speedup vs baseline: 1.1053x; 1.1053x over previous
import jax
import jax.numpy as jnp
import numpy as np
from jax import lax
from jax.experimental import pallas as pl
from jax.experimental.pallas import tpu as pltpu

N_BINS = 15
N_ROWS = 50000
N_COLS = 1000
COL_BLOCK = 4608
GRID = (N_ROWS + COL_BLOCK - 1) // COL_BLOCK

_BOUNDS = np.array(
    [0x00000000, 0x3D888889, 0x3E088889, 0x3E4CCCCE, 0x3E888889, 0x3EAAAAAB,
     0x3ECCCCCE, 0x3EEEEEF0, 0x3F088889, 0x3F19999A, 0x3F2AAAAB, 0x3F3BBBBC,
     0x3F4CCCCE, 0x3F5DDDDF, 0x3F6EEEF0, 0x3F800000],
    dtype=np.uint32,
).view(np.float32)


def _ece_body(x_ref, lbl_ref, out_ref, cnt_ref, csum_ref, asum_ref):
    step = pl.program_id(0)

    @pl.when(step == 0)
    def _init():
        cnt_ref[...] = jnp.zeros_like(cnt_ref)
        csum_ref[...] = jnp.zeros_like(csum_ref)
        asum_ref[...] = jnp.zeros_like(asum_ref)

    x = x_ref[...]
    m = jnp.max(x, axis=0, keepdims=True)
    s = jnp.sum(jnp.exp(x - m), axis=0, keepdims=True)
    conf = 1.0 / s

    ii = lax.broadcasted_iota(jnp.int32, x.shape, 0)
    pred = jnp.min(jnp.where(x == m, ii, N_COLS), axis=0, keepdims=True)
    acc = (pred == lbl_ref[...]).astype(jnp.float32)

    subl = lax.broadcasted_iota(jnp.int32, (2 * 8, 1), 0)
    lo = jnp.full((2 * 8, 1), jnp.inf, jnp.float32)
    up = jnp.full((2 * 8, 1), jnp.inf, jnp.float32)
    for i in range(N_BINS):
        lo = jnp.where(subl == i, _BOUNDS[i], lo)
        up = jnp.where(subl == i, _BOUNDS[i + 1], up)

    sidx = step * COL_BLOCK + lax.broadcasted_iota(jnp.int32, (1, COL_BLOCK), 1)
    valid = sidx < N_ROWS

    confb = jnp.broadcast_to(conf, (2 * 8, COL_BLOCK))
    accb = jnp.broadcast_to(acc, (2 * 8, COL_BLOCK))
    inb = (confb > lo) & (confb <= up) & valid
    zero = jnp.zeros((2 * 8, COL_BLOCK), jnp.float32)
    cnt_ref[...] += jnp.where(inb, 1.0, zero)
    csum_ref[...] += jnp.where(inb, confb, zero)
    asum_ref[...] += jnp.where(inb, accb, zero)

    @pl.when(step == GRID - 1)
    def _combine():
        tot_cnt = jnp.sum(cnt_ref[...], axis=1, keepdims=True)
        tot_conf = jnp.sum(csum_ref[...], axis=1, keepdims=True)
        tot_acc = jnp.sum(asum_ref[...], axis=1, keepdims=True)
        safe = jnp.maximum(tot_cnt, 1.0)
        contrib = jnp.abs(tot_conf / safe - tot_acc / safe) * (
            tot_cnt * (1.0 / N_ROWS)
        )
        contrib = jnp.where(tot_cnt > 0.0, contrib, 0.0)
        out_ref[...] = jnp.broadcast_to(jnp.sum(contrib), (8, 128))


@jax.jit
def kernel(logits, labels):
    xt = logits.T
    lbl = labels.astype(jnp.int32).reshape(1, N_ROWS)
    out = pl.pallas_call(
        _ece_body,
        grid=(GRID,),
        in_specs=[
            pl.BlockSpec((N_COLS, COL_BLOCK), lambda i: (0, i)),
            pl.BlockSpec((1, COL_BLOCK), lambda i: (0, i)),
        ],
        out_specs=pl.BlockSpec((8, 128), lambda i: (0, 0)),
        out_shape=jax.ShapeDtypeStruct((8, 128), jnp.float32),
        scratch_shapes=[
            pltpu.VMEM((2 * 8, COL_BLOCK), jnp.float32),
            pltpu.VMEM((2 * 8, COL_BLOCK), jnp.float32),
            pltpu.VMEM((2 * 8, COL_BLOCK), jnp.float32),
        ],
        compiler_params=pltpu.CompilerParams(
            dimension_semantics=("arbitrary",),
        ),
    )(xt, lbl)
    return out[0, 0].reshape(1)
